# baseline (speedup 1.0000x reference)
import jax
import jax.numpy as jnp
from jax import lax
from jax.experimental import pallas as pl
from jax.experimental.pallas import tpu as pltpu
from jax.experimental.pallas import tpu_sc as plsc

NUM_CLASSES = 1000
DIM = 64
BATCH = 16384
LANES = 16

_info = plsc.get_sparse_core_info()
NUM_CORES = _info.num_cores
NUM_SUBCORES = _info.num_subcores
NW = NUM_CORES * NUM_SUBCORES
B_PER_W = BATCH // NW


def _freq_bias_body(sbj_hbm, obj_hbm, table_hbm, out_hbm,
                    sbj_v, obj_v, outb_v, sem):
    wid = lax.axis_index("s") * NUM_CORES + lax.axis_index("c")
    base = wid * B_PER_W
    pltpu.sync_copy(sbj_hbm.at[pl.ds(base, B_PER_W)], sbj_v)
    pltpu.sync_copy(obj_hbm.at[pl.ds(base, B_PER_W)], obj_v)

    for g in range(B_PER_W // LANES):
        s = sbj_v[pl.ds(g * LANES, LANES)]
        o = obj_v[pl.ds(g * LANES, LANES)]
        f_vec = s * NUM_CLASSES + o
        for l in range(LANES):
            pltpu.async_copy(
                table_hbm.at[f_vec[l]],
                outb_v.at[g * LANES + l],
                sem)

    def drain_body(i, _):
        pltpu.make_async_copy(table_hbm.at[0], outb_v.at[0], sem).wait()
        return _

    lax.fori_loop(0, B_PER_W, drain_body, None)

    pltpu.sync_copy(outb_v, out_hbm.at[pl.ds(base, B_PER_W)])


def kernel(sbj_labels, obj_labels, node_baseline):
    mesh = plsc.VectorSubcoreMesh(core_axis_name="c", subcore_axis_name="s")
    k = pl.kernel(
        _freq_bias_body,
        mesh=mesh,
        compiler_params=pltpu.CompilerParams(use_tc_tiling_on_sc=True),
        out_type=jax.ShapeDtypeStruct((BATCH, DIM), jnp.float32),
        scratch_types=[
            pltpu.VMEM((B_PER_W,), jnp.int32),
            pltpu.VMEM((B_PER_W,), jnp.int32),
            pltpu.VMEM((B_PER_W, DIM), jnp.float32),
            pltpu.SemaphoreType.DMA,
        ],
    )
    return k(sbj_labels.astype(jnp.int32), obj_labels.astype(jnp.int32),
             node_baseline)
